# Initial kernel scaffold; baseline (speedup 1.0000x reference)
import functools

import jax
import jax.numpy as jnp
import numpy as np
from jax.experimental import pallas as pl
from jax.experimental.pallas import tpu as pltpu

GRID_SIZE = 32
M = GRID_SIZE * GRID_SIZE
BOUNDS = (-1.0, 1.0)
N_BLOCK = 512
C_BLOCK = 1024


def _cubic(u):
    u = jnp.abs(u)
    a = ((1.5 * u - 2.5) * u * u + 1.0) * (u <= 1.0)
    b = (((-0.5 * u + 2.5) * u - 4.0) * u + 2.0) * ((u > 1.0) & (u <= 2.0))
    return a + b


def _stage_a_body(rel_ref, chol_ref, vm_ref, t0_ref, t1_ref, a_ref, mean_ref):
    rel0 = rel_ref[:, 0:1]
    rel1 = rel_ref[:, 1:2]
    j = jax.lax.broadcasted_iota(jnp.float32, (1, GRID_SIZE), 1)
    w0 = _cubic(rel0 - j)
    w1 = _cubic(rel1 - j)
    w0big = jnp.dot(w0, t0_ref[...], preferred_element_type=jnp.float32)
    w1big = jnp.dot(w1, t1_ref[...], preferred_element_type=jnp.float32)
    w = w0big * w1big
    mean_ref[...] = jnp.dot(w, vm_ref[...], preferred_element_type=jnp.float32)
    a_ref[...] = jnp.dot(
        w.astype(jnp.bfloat16), chol_ref[...], preferred_element_type=jnp.float32
    ).astype(jnp.bfloat16)


def _covar_body(ai_ref, aj_ref, out_ref):
    i = pl.program_id(0)
    j = pl.program_id(1)
    acc = jax.lax.dot_general(
        ai_ref[...], aj_ref[...], (((1,), (1,)), ((), ())),
        preferred_element_type=jnp.float32,
    )

    @pl.when(i == j)
    def _():
        r = jax.lax.broadcasted_iota(jnp.int32, acc.shape, 0)
        c = jax.lax.broadcasted_iota(jnp.int32, acc.shape, 1)
        out_ref[...] = acc + jnp.where(r == c, jnp.float32(1e-3), jnp.float32(0.0))

    @pl.when(i != j)
    def _():
        out_ref[...] = acc


@jax.jit
def kernel(x, variational_mean, chol_factor):
    n = x.shape[0]
    b0, b1 = BOUNDS
    grid_diff = (b1 - b0) / (GRID_SIZE - 2)
    g0 = b0 - grid_diff
    h = (b1 + grid_diff - g0) / (GRID_SIZE - 1)
    rel = (x - g0) / h

    eye = jnp.eye(GRID_SIZE, dtype=jnp.float32)
    t0 = jnp.tile(eye, (1, GRID_SIZE))
    t1 = jnp.repeat(eye, GRID_SIZE, axis=1)
    chol_bf16 = chol_factor.astype(jnp.bfloat16)
    vm_col = variational_mean.reshape(M, 1)

    a_bf16, mean_col = pl.pallas_call(
        _stage_a_body,
        grid=(n // N_BLOCK,),
        in_specs=[
            pl.BlockSpec((N_BLOCK, 2), lambda i: (i, 0)),
            pl.BlockSpec((M, M), lambda i: (0, 0)),
            pl.BlockSpec((M, 1), lambda i: (0, 0)),
            pl.BlockSpec((GRID_SIZE, M), lambda i: (0, 0)),
            pl.BlockSpec((GRID_SIZE, M), lambda i: (0, 0)),
        ],
        out_specs=[
            pl.BlockSpec((N_BLOCK, M), lambda i: (i, 0)),
            pl.BlockSpec((N_BLOCK, 1), lambda i: (i, 0)),
        ],
        out_shape=[
            jax.ShapeDtypeStruct((n, M), jnp.bfloat16),
            jax.ShapeDtypeStruct((n, 1), jnp.float32),
        ],
    )(rel, chol_bf16, vm_col, t0, t1)

    covar = pl.pallas_call(
        _covar_body,
        grid=(n // C_BLOCK, n // C_BLOCK),
        in_specs=[
            pl.BlockSpec((C_BLOCK, M), lambda i, j: (i, 0)),
            pl.BlockSpec((C_BLOCK, M), lambda i, j: (j, 0)),
        ],
        out_specs=pl.BlockSpec((C_BLOCK, C_BLOCK), lambda i, j: (i, j)),
        out_shape=jax.ShapeDtypeStruct((n, n), jnp.float32),
    )(a_bf16, a_bf16)

    return mean_col.reshape(n), covar


# speedup vs baseline: 3.5724x; 3.5724x over previous
import functools

import jax
import jax.numpy as jnp
import numpy as np
from jax.experimental import pallas as pl
from jax.experimental.pallas import tpu as pltpu

GRID_SIZE = 32
M = GRID_SIZE * GRID_SIZE
BOUNDS = (-1.0, 1.0)
N_BLOCK = 512
C_BLOCK = 1024


def _cubic(u):
    u = jnp.abs(u)
    a = ((1.5 * u - 2.5) * u * u + 1.0) * (u <= 1.0)
    b = (((-0.5 * u + 2.5) * u - 4.0) * u + 2.0) * ((u > 1.0) & (u <= 2.0))
    return a + b


def _stage_a_body(rel_ref, chol_ref, vm_ref, t0_ref, t1_ref, a_ref, mean_ref):
    rel0 = rel_ref[:, 0:1]
    rel1 = rel_ref[:, 1:2]
    j = jax.lax.broadcasted_iota(jnp.int32, (1, GRID_SIZE), 1).astype(jnp.float32)
    w0 = _cubic(rel0 - j)
    w1 = _cubic(rel1 - j)
    w0big = jnp.dot(w0, t0_ref[...], preferred_element_type=jnp.float32)
    w1big = jnp.dot(w1, t1_ref[...], preferred_element_type=jnp.float32)
    w = w0big * w1big
    mean_ref[...] = jnp.dot(w, vm_ref[...], preferred_element_type=jnp.float32)
    a_ref[...] = jnp.dot(
        w.astype(jnp.bfloat16), chol_ref[...], preferred_element_type=jnp.float32
    ).astype(jnp.bfloat16)


def _covar_body(ai_ref, aj_ref, out_ref):
    i = pl.program_id(0)
    j = pl.program_id(1)
    acc = jax.lax.dot_general(
        ai_ref[...], aj_ref[...], (((1,), (1,)), ((), ())),
        preferred_element_type=jnp.float32,
    )

    @pl.when(i == j)
    def _():
        r = jax.lax.broadcasted_iota(jnp.int32, acc.shape, 0)
        c = jax.lax.broadcasted_iota(jnp.int32, acc.shape, 1)
        out_ref[...] = acc + jnp.where(r == c, jnp.float32(1e-3), jnp.float32(0.0))

    @pl.when(i != j)
    def _():
        out_ref[...] = acc


@jax.jit
def kernel(x, variational_mean, chol_factor):
    n = x.shape[0]
    b0, b1 = BOUNDS
    grid_diff = (b1 - b0) / (GRID_SIZE - 2)
    g0 = b0 - grid_diff
    h = (b1 + grid_diff - g0) / (GRID_SIZE - 1)
    rel = (x - g0) / h

    eye = jnp.eye(GRID_SIZE, dtype=jnp.float32)
    t0 = jnp.tile(eye, (1, GRID_SIZE))
    t1 = jnp.repeat(eye, GRID_SIZE, axis=1)
    chol_bf16 = chol_factor.astype(jnp.bfloat16)
    vm_col = variational_mean.reshape(M, 1)

    a_bf16, mean_col = pl.pallas_call(
        _stage_a_body,
        grid=(n // N_BLOCK,),
        in_specs=[
            pl.BlockSpec((N_BLOCK, 2), lambda i: (i, 0)),
            pl.BlockSpec((M, M), lambda i: (0, 0)),
            pl.BlockSpec((M, 1), lambda i: (0, 0)),
            pl.BlockSpec((GRID_SIZE, M), lambda i: (0, 0)),
            pl.BlockSpec((GRID_SIZE, M), lambda i: (0, 0)),
        ],
        out_specs=[
            pl.BlockSpec((N_BLOCK, M), lambda i: (i, 0)),
            pl.BlockSpec((N_BLOCK, 1), lambda i: (i, 0)),
        ],
        out_shape=[
            jax.ShapeDtypeStruct((n, M), jnp.bfloat16),
            jax.ShapeDtypeStruct((n, 1), jnp.float32),
        ],
    )(rel, chol_bf16, vm_col, t0, t1)

    covar = pl.pallas_call(
        _covar_body,
        grid=(n // C_BLOCK, n // C_BLOCK),
        in_specs=[
            pl.BlockSpec((C_BLOCK, M), lambda i, j: (i, 0)),
            pl.BlockSpec((C_BLOCK, M), lambda i, j: (j, 0)),
        ],
        out_specs=pl.BlockSpec((C_BLOCK, C_BLOCK), lambda i, j: (i, j)),
        out_shape=jax.ShapeDtypeStruct((n, n), jnp.float32),
    )(a_bf16, a_bf16)

    return mean_col.reshape(n), covar


# speedup vs baseline: 4.0579x; 1.1359x over previous
import functools

import jax
import jax.numpy as jnp
import numpy as np
from jax.experimental import pallas as pl
from jax.experimental.pallas import tpu as pltpu

GRID_SIZE = 32
M = GRID_SIZE * GRID_SIZE
BOUNDS = (-1.0, 1.0)
N_TOTAL = 4096
N_BLOCK = 512
C_BLOCK = 1024


def _cubic(u):
    u = jnp.abs(u)
    a = ((1.5 * u - 2.5) * u * u + 1.0) * (u <= 1.0)
    b = (((-0.5 * u + 2.5) * u - 4.0) * u + 2.0) * ((u > 1.0) & (u <= 2.0))
    return a + b


def _fused_body(rel_ref, chol_ref, vm_ref, t0_ref, t1_ref,
                covar_ref, mean_ref, a_scr):
    i = pl.program_id(0)
    j = pl.program_id(1)

    @pl.when((i == 0) & (j == 0))
    def _stage_a():
        jcol = jax.lax.broadcasted_iota(jnp.int32, (1, GRID_SIZE), 1).astype(
            jnp.float32)
        for c in range(N_TOTAL // N_BLOCK):
            lo = c * N_BLOCK
            rel0 = rel_ref[lo:lo + N_BLOCK, 0:1]
            rel1 = rel_ref[lo:lo + N_BLOCK, 1:2]
            w0 = _cubic(rel0 - jcol)
            w1 = _cubic(rel1 - jcol)
            w0big = jnp.dot(w0, t0_ref[...], preferred_element_type=jnp.float32)
            w1big = jnp.dot(w1, t1_ref[...], preferred_element_type=jnp.float32)
            w = w0big * w1big
            mean_ref[lo:lo + N_BLOCK, :] = jnp.dot(
                w, vm_ref[...], preferred_element_type=jnp.float32)
            a_scr[lo:lo + N_BLOCK, :] = jnp.dot(
                w.astype(jnp.bfloat16), chol_ref[...],
                preferred_element_type=jnp.float32).astype(jnp.bfloat16)

    ai = a_scr[pl.ds(i * C_BLOCK, C_BLOCK), :]
    aj = a_scr[pl.ds(j * C_BLOCK, C_BLOCK), :]
    acc = jax.lax.dot_general(
        ai, aj, (((1,), (1,)), ((), ())), preferred_element_type=jnp.float32)

    @pl.when(i == j)
    def _():
        r = jax.lax.broadcasted_iota(jnp.int32, acc.shape, 0)
        c = jax.lax.broadcasted_iota(jnp.int32, acc.shape, 1)
        covar_ref[...] = acc + jnp.where(r == c, jnp.float32(1e-3),
                                         jnp.float32(0.0))

    @pl.when(i != j)
    def _():
        covar_ref[...] = acc


@jax.jit
def kernel(x, variational_mean, chol_factor):
    n = x.shape[0]
    b0, b1 = BOUNDS
    grid_diff = (b1 - b0) / (GRID_SIZE - 2)
    g0 = b0 - grid_diff
    h = (b1 + grid_diff - g0) / (GRID_SIZE - 1)
    rel = (x - g0) / h

    eye = jnp.eye(GRID_SIZE, dtype=jnp.float32)
    t0 = jnp.tile(eye, (1, GRID_SIZE))
    t1 = jnp.repeat(eye, GRID_SIZE, axis=1)
    chol_bf16 = chol_factor.astype(jnp.bfloat16)
    vm_col = variational_mean.reshape(M, 1)

    covar, mean_col = pl.pallas_call(
        _fused_body,
        grid=(n // C_BLOCK, n // C_BLOCK),
        in_specs=[
            pl.BlockSpec((n, 2), lambda i, j: (0, 0)),
            pl.BlockSpec((M, M), lambda i, j: (0, 0)),
            pl.BlockSpec((M, 1), lambda i, j: (0, 0)),
            pl.BlockSpec((GRID_SIZE, M), lambda i, j: (0, 0)),
            pl.BlockSpec((GRID_SIZE, M), lambda i, j: (0, 0)),
        ],
        out_specs=[
            pl.BlockSpec((C_BLOCK, C_BLOCK), lambda i, j: (i, j)),
            pl.BlockSpec((n, 1), lambda i, j: (0, 0)),
        ],
        out_shape=[
            jax.ShapeDtypeStruct((n, n), jnp.float32),
            jax.ShapeDtypeStruct((n, 1), jnp.float32),
        ],
        scratch_shapes=[pltpu.VMEM((n, M), jnp.bfloat16)],
    )(rel, chol_bf16, vm_col, t0, t1)

    return mean_col.reshape(n), covar
